# baseline (speedup 1.0000x reference)
import functools

import jax
import jax.numpy as jnp
import numpy as np
from jax import lax
from jax.experimental import pallas as pl
from jax.experimental.pallas import tpu as pltpu
from jax.experimental.pallas import tpu_sc as plsc

_SPATIAL_SCALE = 0.125
_P = 7
_S = 4
_TRANS_STD = 0.1
_PART_SIZE = 7

_NC = 2
_NS = 16
_NW = _NC * _NS
_L = 16

_SAMPLES = _S * _S
_BINS = _P * _P
_ROW_BLK = 128


def _weights_body(rois_ref, offx_ref, offy_ref, idx_ref, w00_ref, w01_ref,
                  w10_ref, w11_ref, *, H, W):
    f32 = jnp.float32
    cols = _BINS * _SAMPLES
    bi = rois_ref[:, 0:1].astype(jnp.int32)
    sw = jnp.round(rois_ref[:, 1:2]) * _SPATIAL_SCALE - 0.5
    sh = jnp.round(rois_ref[:, 2:3]) * _SPATIAL_SCALE - 0.5
    ew = (jnp.round(rois_ref[:, 3:4]) + 1.0) * _SPATIAL_SCALE - 0.5
    eh = (jnp.round(rois_ref[:, 4:5]) + 1.0) * _SPATIAL_SCALE - 0.5
    roi_w = jnp.maximum(ew - sw, 0.1)
    roi_h = jnp.maximum(eh - sh, 0.1)
    bin_w = roi_w / _P
    bin_h = roi_h / _P
    sub_w = bin_w / _S
    sub_h = bin_h / _S

    ci = lax.broadcasted_iota(jnp.int32, (_ROW_BLK, cols), 1)
    binc = ci // _SAMPLES
    s = ci - binc * _SAMPLES
    ih = s // _S
    iw = s - ih * _S
    phv = binc // _P
    pwv = binc - phv * _P
    phf = phv.astype(f32)
    pwf = pwv.astype(f32)
    ihf = ih.astype(f32)
    iwf = iw.astype(f32)

    tx = offx_ref[...] * _TRANS_STD
    ty = offy_ref[...] * _TRANS_STD
    wstart = pwf * bin_w + sw + tx * roi_w
    hstart = phf * bin_h + sh + ty * roi_h
    w = wstart + iwf * sub_w
    h = hstart + ihf * sub_h

    def _inrange(v, lim):
        return (v >= -0.5) & (v <= lim - 0.5)

    valid = (_inrange(w, W) & _inrange(h, H)).astype(f32)
    cw = sum(_inrange(wstart + float(j) * sub_w, W).astype(f32)
             for j in range(_S))
    ch = sum(_inrange(hstart + float(j) * sub_h, H).astype(f32)
             for j in range(_S))
    cnt = cw * ch

    wc = jnp.clip(w, 0.0, W - 1.0)
    hc = jnp.clip(h, 0.0, H - 1.0)
    x0 = jnp.floor(wc)
    y0 = jnp.floor(hc)
    dx = wc - x0
    dy = hc - y0
    idx_ref[...] = bi * (H * W) + y0.astype(jnp.int32) * W + x0.astype(jnp.int32)
    scale = valid / jnp.maximum(cnt, 1.0)
    w00_ref[...] = (1.0 - dx) * (1.0 - dy) * scale
    w01_ref[...] = dx * (1.0 - dy) * scale
    w10_ref[...] = (1.0 - dx) * dy * scale
    w11_ref[...] = dx * dy * scale


def _compute_weights(rois_p, offx, offy, H, W):
    NP = rois_p.shape[0]
    cols = _BINS * _SAMPLES
    grid = (NP // _ROW_BLK,)
    spec_r = pl.BlockSpec((_ROW_BLK, 5), lambda i: (i, 0))
    spec_c = pl.BlockSpec((_ROW_BLK, cols), lambda i: (i, 0))
    out_shapes = [jax.ShapeDtypeStruct((NP, cols), jnp.int32)] + \
                 [jax.ShapeDtypeStruct((NP, cols), jnp.float32)] * 4
    return pl.pallas_call(
        functools.partial(_weights_body, H=H, W=W),
        grid=grid,
        in_specs=[spec_r, spec_c, spec_c],
        out_specs=[spec_c] * 5,
        out_shape=out_shapes,
    )(rois_p, offx, offy)


def _sc_pool(tab, idx_flat, wts_flat, n_bins_padded, C):
    bpw = n_bins_padded // _NW
    chunk = 32
    nchunk = bpw // chunk
    ncc = C // _L
    _D = 4
    mesh = plsc.VectorSubcoreMesh(core_axis_name="c", subcore_axis_name="s",
                                  num_cores=_NC, num_subcores=_NS)

    @functools.partial(
        pl.kernel,
        mesh=mesh,
        out_type=jax.ShapeDtypeStruct((n_bins_padded * C,), jnp.float32),
        scratch_types=[
            pltpu.VMEM((chunk * _SAMPLES,), jnp.int32),
            pltpu.VMEM((chunk * _SAMPLES * 4,), jnp.float32),
            pltpu.VMEM((_D, _SAMPLES, 4 * C), jnp.float32),
            pltpu.VMEM((_D, C), jnp.float32),
        ] + [pltpu.SemaphoreType.DMA] * (2 * _D),
    )
    def body(tab_hbm, idx_hbm, wts_hbm, out_hbm, idx_v, wts_v, rows_v,
             out_v, *sems):
        wid = lax.axis_index("s") * _NC + lax.axis_index("c")
        bin0 = wid * bpw
        gsems = sems[:_D]
        osems = sems[_D:]

        def gather(b, buf):
            return pltpu.async_copy(
                tab_hbm.at[idx_v.at[pl.ds(b * _SAMPLES, _SAMPLES)]],
                rows_v.at[buf], gsems[buf])

        def wait_gather(buf):
            pltpu.make_async_copy(
                tab_hbm.at[idx_v.at[pl.ds(0, _SAMPLES)]],
                rows_v.at[buf], gsems[buf]).wait()

        def accumulate(cb, b, buf, seen_prior):
            G = ncc
            wvecs = [wts_v[pl.ds(b * (_SAMPLES * 4) + g * _L, _L)]
                     for g in range(_SAMPLES * 4 // _L)]
            @pl.when(seen_prior)
            def _():
                pltpu.make_async_copy(
                    out_v.at[buf], out_hbm.at[pl.ds(0, C)],
                    osems[buf]).wait()
            for c0 in range(0, ncc, G):
                accs = [jnp.zeros((_L,), jnp.float32) for _ in range(G)]
                for k in range(_SAMPLES):
                    for q in range(4):
                        j = k * 4 + q
                        wsp = lax.broadcast(wvecs[j // _L][j % _L], (_L,))
                        for c in range(G):
                            accs[c] = accs[c] + wsp * rows_v[
                                buf, k, pl.ds(q * C + (c0 + c) * _L, _L)]
                for c in range(G):
                    out_v[buf, pl.ds((c0 + c) * _L, _L)] = accs[c]
            pltpu.async_copy(out_v.at[buf],
                             out_hbm.at[pl.ds((cb + b) * C, C)], osems[buf])

        nq = chunk // _D

        @pl.loop(0, nchunk)
        def _chunk_loop(chi):
            cb = bin0 + chi * chunk
            pltpu.sync_copy(idx_hbm.at[pl.ds(cb * _SAMPLES, chunk * _SAMPLES)],
                            idx_v)
            pltpu.sync_copy(
                wts_hbm.at[pl.ds(cb * _SAMPLES * 4, chunk * _SAMPLES * 4)],
                wts_v)
            for buf in range(_D - 1):
                gather(buf, buf)

            @pl.loop(0, nq)
            def _quad_loop(p):
                seen = jnp.logical_or(chi > 0, p > 0)
                b = p * _D
                gather(b + _D - 1, _D - 1)
                for u in range(_D):
                    wait_gather(u)
                    accumulate(cb, b + u, u, seen)
                    if u < _D - 1:
                        @pl.when(p < nq - 1)
                        def _():
                            gather(b + _D + u, u)

        for buf in range(_D):
            pltpu.make_async_copy(out_v.at[buf], out_hbm.at[pl.ds(0, C)],
                                  osems[buf]).wait()

    return body(tab, idx_flat, wts_flat)


def kernel(input, rois, offset):
    B, C, H, W = input.shape
    N = rois.shape[0]
    NP = ((N + _ROW_BLK - 1) // _ROW_BLK) * _ROW_BLK
    nbins = NP * _BINS
    V = B * H * W

    xt = jnp.transpose(input, (0, 2, 3, 1)).reshape(V, C)
    xtp = jnp.pad(xt, ((0, W + 2), (0, 0)))
    tab = jnp.concatenate(
        [xtp[0:V], xtp[1:V + 1], xtp[W:V + W], xtp[W + 1:V + W + 1]], axis=1)

    pr = np.arange(_P, dtype=np.float32)
    part = np.floor(pr / np.float32(_P) * np.float32(_PART_SIZE)).astype(np.int64)
    off_x = offset[:, 0][:, part, :][:, :, part].reshape(N, _BINS)
    off_y = offset[:, 1][:, part, :][:, :, part].reshape(N, _BINS)
    offx = jnp.repeat(off_x, _SAMPLES, axis=1)
    offy = jnp.repeat(off_y, _SAMPLES, axis=1)

    rois_p = jnp.pad(rois, ((0, NP - N), (0, 0)))
    offx = jnp.pad(offx, ((0, NP - N), (0, 0)))
    offy = jnp.pad(offy, ((0, NP - N), (0, 0)))

    idx, w00, w01, w10, w11 = _compute_weights(rois_p, offx, offy, H, W)

    idx_flat = idx.reshape(nbins * _SAMPLES)
    wts_flat = jnp.stack([w00, w01, w10, w11], axis=-1).reshape(
        nbins * _SAMPLES * 4)

    out_flat = _sc_pool(tab, idx_flat, wts_flat, nbins, C)

    out = out_flat.reshape(NP, _BINS, C)[:N]
    out = out.reshape(N, _P, _P, C)
    return jnp.transpose(out, (0, 3, 1, 2))


# speedup vs baseline: 2.1471x; 2.1471x over previous
import functools

import jax
import jax.numpy as jnp
import numpy as np
from jax import lax
from jax.experimental import pallas as pl
from jax.experimental.pallas import tpu as pltpu
from jax.experimental.pallas import tpu_sc as plsc

_SPATIAL_SCALE = 0.125
_P = 7
_S = 4
_TRANS_STD = 0.1
_PART_SIZE = 7

_NC = 2
_NS = 16
_NW = _NC * _NS
_L = 16

_SAMPLES = _S * _S
_BINS = _P * _P
_ROW_BLK = 128


def _weights_body(rois_ref, offx_ref, offy_ref, idx_ref, w00_ref, w01_ref,
                  w10_ref, w11_ref, *, H, W):
    f32 = jnp.float32
    cols = _BINS * _SAMPLES
    bi = rois_ref[:, 0:1].astype(jnp.int32)
    sw = jnp.round(rois_ref[:, 1:2]) * _SPATIAL_SCALE - 0.5
    sh = jnp.round(rois_ref[:, 2:3]) * _SPATIAL_SCALE - 0.5
    ew = (jnp.round(rois_ref[:, 3:4]) + 1.0) * _SPATIAL_SCALE - 0.5
    eh = (jnp.round(rois_ref[:, 4:5]) + 1.0) * _SPATIAL_SCALE - 0.5
    roi_w = jnp.maximum(ew - sw, 0.1)
    roi_h = jnp.maximum(eh - sh, 0.1)
    bin_w = roi_w / _P
    bin_h = roi_h / _P
    sub_w = bin_w / _S
    sub_h = bin_h / _S

    ci = lax.broadcasted_iota(jnp.int32, (_ROW_BLK, cols), 1)
    binc = ci // _SAMPLES
    s = ci - binc * _SAMPLES
    ih = s // _S
    iw = s - ih * _S
    phv = binc // _P
    pwv = binc - phv * _P
    phf = phv.astype(f32)
    pwf = pwv.astype(f32)
    ihf = ih.astype(f32)
    iwf = iw.astype(f32)

    tx = offx_ref[...] * _TRANS_STD
    ty = offy_ref[...] * _TRANS_STD
    wstart = pwf * bin_w + sw + tx * roi_w
    hstart = phf * bin_h + sh + ty * roi_h
    w = wstart + iwf * sub_w
    h = hstart + ihf * sub_h

    def _inrange(v, lim):
        return (v >= -0.5) & (v <= lim - 0.5)

    valid = (_inrange(w, W) & _inrange(h, H)).astype(f32)
    cw = sum(_inrange(wstart + float(j) * sub_w, W).astype(f32)
             for j in range(_S))
    ch = sum(_inrange(hstart + float(j) * sub_h, H).astype(f32)
             for j in range(_S))
    cnt = cw * ch

    wc = jnp.clip(w, 0.0, W - 1.0)
    hc = jnp.clip(h, 0.0, H - 1.0)
    x0 = jnp.floor(wc)
    y0 = jnp.floor(hc)
    dx = wc - x0
    dy = hc - y0
    idx_ref[...] = bi * (H * W) + y0.astype(jnp.int32) * W + x0.astype(jnp.int32)
    scale = valid / jnp.maximum(cnt, 1.0)
    w00_ref[...] = (1.0 - dx) * (1.0 - dy) * scale
    w01_ref[...] = dx * (1.0 - dy) * scale
    w10_ref[...] = (1.0 - dx) * dy * scale
    w11_ref[...] = dx * dy * scale


def _compute_weights(rois_p, offx, offy, H, W):
    NP = rois_p.shape[0]
    cols = _BINS * _SAMPLES
    grid = (NP // _ROW_BLK,)
    spec_r = pl.BlockSpec((_ROW_BLK, 5), lambda i: (i, 0))
    spec_c = pl.BlockSpec((_ROW_BLK, cols), lambda i: (i, 0))
    out_shapes = [jax.ShapeDtypeStruct((NP, cols), jnp.int32)] + \
                 [jax.ShapeDtypeStruct((NP, cols), jnp.float32)] * 4
    return pl.pallas_call(
        functools.partial(_weights_body, H=H, W=W),
        grid=grid,
        in_specs=[spec_r, spec_c, spec_c],
        out_specs=[spec_c] * 5,
        out_shape=out_shapes,
    )(rois_p, offx, offy)


def _sc_pool(tab, idx_flat, wts_flat, n_bins_padded, C):
    bpw = n_bins_padded // _NW
    chunk = 32
    nchunk = bpw // chunk
    ncc = C // _L
    _D = 4
    mesh = plsc.VectorSubcoreMesh(core_axis_name="c", subcore_axis_name="s",
                                  num_cores=_NC, num_subcores=_NS)

    @functools.partial(
        pl.kernel,
        mesh=mesh,
        out_type=jax.ShapeDtypeStruct((n_bins_padded * C,), jnp.float32),
        scratch_types=[
            pltpu.VMEM((chunk * _SAMPLES,), jnp.int32),
            pltpu.VMEM((chunk * _SAMPLES * 4,), jnp.float32),
            pltpu.VMEM((_D, _SAMPLES, 4 * C), jnp.float32),
            pltpu.VMEM((_D, C), jnp.float32),
        ] + [pltpu.SemaphoreType.DMA] * (2 * _D),
    )
    def body(tab_hbm, idx_hbm, wts_hbm, out_hbm, idx_v, wts_v, rows_v,
             out_v, *sems):
        wid = lax.axis_index("s") * _NC + lax.axis_index("c")
        bin0 = wid * bpw
        gsems = sems[:_D]
        osems = sems[_D:]

        def gather(b, buf):
            return pltpu.async_copy(
                tab_hbm.at[idx_v.at[pl.ds(b * _SAMPLES, _SAMPLES)]],
                rows_v.at[buf], gsems[buf])

        def wait_gather(buf):
            pltpu.make_async_copy(
                tab_hbm.at[idx_v.at[pl.ds(0, _SAMPLES)]],
                rows_v.at[buf], gsems[buf]).wait()

        def accumulate(cb, b, buf, seen_prior):
            @pl.when(seen_prior)
            def _():
                pltpu.make_async_copy(
                    out_v.at[buf], out_hbm.at[pl.ds(0, C)],
                    osems[buf]).wait()
            zero = jnp.zeros((_L,), jnp.float32)
            for c in range(ncc):
                out_v[buf, pl.ds(c * _L, _L)] = zero

            @pl.loop(0, 4)
            def _grp(t):
                wvec = wts_v[pl.ds(b * (_SAMPLES * 4) + t * _L, _L)]
                wsps = [lax.broadcast(wvec[dj], (_L,)) for dj in range(16)]
                k4 = t * 4

                def term(dj, c):
                    return wsps[dj] * rows_v[
                        buf, k4 + dj // 4,
                        pl.ds((dj % 4) * C + c * _L, _L)]

                for c0 in range(0, ncc, 2):
                    subs = [[term(dj, c0 + h) for dj in range(4)]
                            for h in range(2)]
                    for dj in range(4, 16):
                        for h in range(2):
                            subs[h][dj % 4] = subs[h][dj % 4] + term(
                                dj, c0 + h)
                    for h in range(2):
                        acc = (subs[h][0] + subs[h][1]) + (
                            subs[h][2] + subs[h][3])
                        plsc.addupdate(
                            out_v.at[buf, pl.ds((c0 + h) * _L, _L)], acc)
            pltpu.async_copy(out_v.at[buf],
                             out_hbm.at[pl.ds((cb + b) * C, C)], osems[buf])

        nq = chunk // _D

        @pl.loop(0, nchunk)
        def _chunk_loop(chi):
            cb = bin0 + chi * chunk
            pltpu.sync_copy(idx_hbm.at[pl.ds(cb * _SAMPLES, chunk * _SAMPLES)],
                            idx_v)
            pltpu.sync_copy(
                wts_hbm.at[pl.ds(cb * _SAMPLES * 4, chunk * _SAMPLES * 4)],
                wts_v)
            for buf in range(_D - 1):
                gather(buf, buf)

            @pl.loop(0, nq)
            def _quad_loop(p):
                seen = jnp.logical_or(chi > 0, p > 0)
                b = p * _D
                gather(b + _D - 1, _D - 1)
                for u in range(_D):
                    wait_gather(u)
                    accumulate(cb, b + u, u, seen)
                    if u < _D - 1:
                        @pl.when(p < nq - 1)
                        def _():
                            gather(b + _D + u, u)

        for buf in range(_D):
            pltpu.make_async_copy(out_v.at[buf], out_hbm.at[pl.ds(0, C)],
                                  osems[buf]).wait()

    return body(tab, idx_flat, wts_flat)


def kernel(input, rois, offset):
    B, C, H, W = input.shape
    N = rois.shape[0]
    NP = ((N + _ROW_BLK - 1) // _ROW_BLK) * _ROW_BLK
    nbins = NP * _BINS
    V = B * H * W

    xt = jnp.transpose(input, (0, 2, 3, 1)).reshape(V, C)
    xtp = jnp.pad(xt, ((0, W + 2), (0, 0)))
    tab = jnp.concatenate(
        [xtp[0:V], xtp[1:V + 1], xtp[W:V + W], xtp[W + 1:V + W + 1]], axis=1)

    pr = np.arange(_P, dtype=np.float32)
    part = np.floor(pr / np.float32(_P) * np.float32(_PART_SIZE)).astype(np.int64)
    off_x = offset[:, 0][:, part, :][:, :, part].reshape(N, _BINS)
    off_y = offset[:, 1][:, part, :][:, :, part].reshape(N, _BINS)
    offx = jnp.repeat(off_x, _SAMPLES, axis=1)
    offy = jnp.repeat(off_y, _SAMPLES, axis=1)

    rois_p = jnp.pad(rois, ((0, NP - N), (0, 0)))
    offx = jnp.pad(offx, ((0, NP - N), (0, 0)))
    offy = jnp.pad(offy, ((0, NP - N), (0, 0)))

    idx, w00, w01, w10, w11 = _compute_weights(rois_p, offx, offy, H, W)

    idx_flat = idx.reshape(nbins * _SAMPLES)
    wts_flat = jnp.stack([w00, w01, w10, w11], axis=-1).reshape(
        nbins * _SAMPLES * 4)

    out_flat = _sc_pool(tab, idx_flat, wts_flat, nbins, C)

    out = out_flat.reshape(NP, _BINS, C)[:N]
    out = out.reshape(N, _P, _P, C)
    return jnp.transpose(out, (0, 3, 1, 2))


# speedup vs baseline: 2.1640x; 1.0079x over previous
import functools

import jax
import jax.numpy as jnp
import numpy as np
from jax import lax
from jax.experimental import pallas as pl
from jax.experimental.pallas import tpu as pltpu
from jax.experimental.pallas import tpu_sc as plsc

_SPATIAL_SCALE = 0.125
_P = 7
_S = 4
_TRANS_STD = 0.1
_PART_SIZE = 7

_NC = 2
_NS = 16
_NW = _NC * _NS
_L = 16

_SAMPLES = _S * _S
_BINS = _P * _P
_ROW_BLK = 128


def _coeffs(rois_p, offx, offy, H, W):
    f32 = jnp.float32
    cols = _BINS * _SAMPLES
    NP = rois_p.shape[0]
    bi = rois_p[:, 0:1].astype(jnp.int32)
    sw = jnp.round(rois_p[:, 1:2]) * _SPATIAL_SCALE - 0.5
    sh = jnp.round(rois_p[:, 2:3]) * _SPATIAL_SCALE - 0.5
    ew = (jnp.round(rois_p[:, 3:4]) + 1.0) * _SPATIAL_SCALE - 0.5
    eh = (jnp.round(rois_p[:, 4:5]) + 1.0) * _SPATIAL_SCALE - 0.5
    roi_w = jnp.maximum(ew - sw, 0.1)
    roi_h = jnp.maximum(eh - sh, 0.1)
    bin_w = roi_w / _P
    bin_h = roi_h / _P
    sub_w = bin_w / _S
    sub_h = bin_h / _S

    ci = lax.broadcasted_iota(jnp.int32, (NP, cols), 1)
    binc = ci // _SAMPLES
    s = ci - binc * _SAMPLES
    ih = s // _S
    iw = s - ih * _S
    phv = binc // _P
    pwv = binc - phv * _P

    tx = offx * _TRANS_STD
    ty = offy * _TRANS_STD
    wstart = pwv.astype(f32) * bin_w + sw + tx * roi_w
    hstart = phv.astype(f32) * bin_h + sh + ty * roi_h
    w = wstart + iw.astype(f32) * sub_w
    h = hstart + ih.astype(f32) * sub_h

    def _inrange(v, lim):
        return (v >= -0.5) & (v <= lim - 0.5)

    valid = (_inrange(w, W) & _inrange(h, H)).astype(f32)
    cw = sum(_inrange(wstart + float(j) * sub_w, W).astype(f32)
             for j in range(_S))
    ch = sum(_inrange(hstart + float(j) * sub_h, H).astype(f32)
             for j in range(_S))
    cnt = cw * ch

    wc = jnp.clip(w, 0.0, W - 1.0)
    hc = jnp.clip(h, 0.0, H - 1.0)
    x0 = jnp.floor(wc)
    y0 = jnp.floor(hc)
    dx = wc - x0
    dy = hc - y0
    idx = bi * (H * W) + y0.astype(jnp.int32) * W + x0.astype(jnp.int32)
    scale = valid / jnp.maximum(cnt, 1.0)
    w00 = (1.0 - dx) * (1.0 - dy) * scale
    w01 = dx * (1.0 - dy) * scale
    w10 = (1.0 - dx) * dy * scale
    w11 = dx * dy * scale
    return idx, w00, w01, w10, w11


def _sc_pool(tab, idx_flat, wts_flat, n_bins_padded, C):
    bpw = n_bins_padded // _NW
    chunk = 32
    nchunk = bpw // chunk
    ncc = C // _L
    _D = 4
    mesh = plsc.VectorSubcoreMesh(core_axis_name="c", subcore_axis_name="s",
                                  num_cores=_NC, num_subcores=_NS)

    @functools.partial(
        pl.kernel,
        mesh=mesh,
        out_type=jax.ShapeDtypeStruct((n_bins_padded * C,), jnp.float32),
        scratch_types=[
            pltpu.VMEM((chunk * _SAMPLES,), jnp.int32),
            pltpu.VMEM((chunk * _SAMPLES * 4,), jnp.float32),
            pltpu.VMEM((_D, _SAMPLES, 4 * C), jnp.float32),
            pltpu.VMEM((_D, C), jnp.float32),
        ] + [pltpu.SemaphoreType.DMA] * (2 * _D),
    )
    def body(tab_hbm, idx_hbm, wts_hbm, out_hbm, idx_v, wts_v, rows_v,
             out_v, *sems):
        wid = lax.axis_index("s") * _NC + lax.axis_index("c")
        bin0 = wid * bpw
        gsems = sems[:_D]
        osems = sems[_D:]

        def gather(b, buf):
            return pltpu.async_copy(
                tab_hbm.at[idx_v.at[pl.ds(b * _SAMPLES, _SAMPLES)]],
                rows_v.at[buf], gsems[buf])

        def wait_gather(buf):
            pltpu.make_async_copy(
                tab_hbm.at[idx_v.at[pl.ds(0, _SAMPLES)]],
                rows_v.at[buf], gsems[buf]).wait()

        def accumulate(cb, b, buf, seen_prior):
            @pl.when(seen_prior)
            def _():
                pltpu.make_async_copy(
                    out_v.at[buf], out_hbm.at[pl.ds(0, C)],
                    osems[buf]).wait()
            zero = jnp.zeros((_L,), jnp.float32)
            for c in range(ncc):
                out_v[buf, pl.ds(c * _L, _L)] = zero

            @pl.loop(0, 4)
            def _grp(t):
                wvec = wts_v[pl.ds(b * (_SAMPLES * 4) + t * _L, _L)]
                wsps = [lax.broadcast(wvec[dj], (_L,)) for dj in range(16)]
                k4 = t * 4

                def term(dj, c):
                    return wsps[dj] * rows_v[
                        buf, k4 + dj // 4,
                        pl.ds((dj % 4) * C + c * _L, _L)]

                for c0 in range(0, ncc, 2):
                    subs = [[term(dj, c0 + h) for dj in range(4)]
                            for h in range(2)]
                    for dj in range(4, 16):
                        for h in range(2):
                            subs[h][dj % 4] = subs[h][dj % 4] + term(
                                dj, c0 + h)
                    for h in range(2):
                        acc = (subs[h][0] + subs[h][1]) + (
                            subs[h][2] + subs[h][3])
                        plsc.addupdate(
                            out_v.at[buf, pl.ds((c0 + h) * _L, _L)], acc)
            pltpu.async_copy(out_v.at[buf],
                             out_hbm.at[pl.ds((cb + b) * C, C)], osems[buf])

        nq = chunk // _D

        @pl.loop(0, nchunk)
        def _chunk_loop(chi):
            cb = bin0 + chi * chunk
            pltpu.sync_copy(idx_hbm.at[pl.ds(cb * _SAMPLES, chunk * _SAMPLES)],
                            idx_v)
            pltpu.sync_copy(
                wts_hbm.at[pl.ds(cb * _SAMPLES * 4, chunk * _SAMPLES * 4)],
                wts_v)
            for buf in range(_D - 1):
                gather(buf, buf)

            @pl.loop(0, nq)
            def _quad_loop(p):
                seen = jnp.logical_or(chi > 0, p > 0)
                b = p * _D
                gather(b + _D - 1, _D - 1)
                for u in range(_D):
                    wait_gather(u)
                    accumulate(cb, b + u, u, seen)
                    if u < _D - 1:
                        @pl.when(p < nq - 1)
                        def _():
                            gather(b + _D + u, u)

        for buf in range(_D):
            pltpu.make_async_copy(out_v.at[buf], out_hbm.at[pl.ds(0, C)],
                                  osems[buf]).wait()

    return body(tab, idx_flat, wts_flat)


def kernel(input, rois, offset):
    B, C, H, W = input.shape
    N = rois.shape[0]
    NP = ((N + _ROW_BLK - 1) // _ROW_BLK) * _ROW_BLK
    nbins = NP * _BINS
    V = B * H * W

    xt = jnp.transpose(input, (0, 2, 3, 1)).reshape(V, C)
    xtp = jnp.pad(xt, ((0, W + 2), (0, 0)))
    tab = jnp.concatenate(
        [xtp[0:V], xtp[1:V + 1], xtp[W:V + W], xtp[W + 1:V + W + 1]], axis=1)

    pr = np.arange(_P, dtype=np.float32)
    part = np.floor(pr / np.float32(_P) * np.float32(_PART_SIZE)).astype(np.int64)
    off_x = offset[:, 0][:, part, :][:, :, part].reshape(N, _BINS)
    off_y = offset[:, 1][:, part, :][:, :, part].reshape(N, _BINS)
    offx = jnp.repeat(off_x, _SAMPLES, axis=1)
    offy = jnp.repeat(off_y, _SAMPLES, axis=1)

    rois_p = jnp.pad(rois, ((0, NP - N), (0, 0)))
    offx = jnp.pad(offx, ((0, NP - N), (0, 0)))
    offy = jnp.pad(offy, ((0, NP - N), (0, 0)))

    idx, w00, w01, w10, w11 = _coeffs(rois_p, offx, offy, H, W)

    idx_flat = idx.reshape(nbins * _SAMPLES)
    wts_flat = jnp.stack([w00, w01, w10, w11], axis=-1).reshape(
        nbins * _SAMPLES * 4)

    out_flat = _sc_pool(tab, idx_flat, wts_flat, nbins, C)

    out = out_flat.reshape(NP, _BINS, C)[:N]
    out = out.reshape(N, _P, _P, C)
    return jnp.transpose(out, (0, 3, 1, 2))


# speedup vs baseline: 2.7811x; 1.2852x over previous
import functools

import jax
import jax.numpy as jnp
import numpy as np
from jax import lax
from jax.experimental import pallas as pl
from jax.experimental.pallas import tpu as pltpu
from jax.experimental.pallas import tpu_sc as plsc

_SPATIAL_SCALE = 0.125
_P = 7
_S = 4
_TRANS_STD = 0.1
_PART_SIZE = 7

_NC = 2
_NS = 16
_NW = _NC * _NS
_L = 16

_SAMPLES = _S * _S
_BINS = _P * _P
_ROW_BLK = 128


def _bin_meta(rois_p, offx, offy, H, W):
    f32 = jnp.float32
    NP = rois_p.shape[0]
    bi = rois_p[:, 0:1].astype(jnp.int32).astype(f32)
    sw = jnp.round(rois_p[:, 1:2]) * _SPATIAL_SCALE - 0.5
    sh = jnp.round(rois_p[:, 2:3]) * _SPATIAL_SCALE - 0.5
    ew = (jnp.round(rois_p[:, 3:4]) + 1.0) * _SPATIAL_SCALE - 0.5
    eh = (jnp.round(rois_p[:, 4:5]) + 1.0) * _SPATIAL_SCALE - 0.5
    roi_w = jnp.maximum(ew - sw, 0.1)
    roi_h = jnp.maximum(eh - sh, 0.1)
    bin_w = roi_w / _P
    bin_h = roi_h / _P
    sub_w = bin_w / _S
    sub_h = bin_h / _S

    binc = lax.broadcasted_iota(jnp.int32, (NP, _BINS), 1)
    phv = binc // _P
    pwv = binc - phv * _P

    wstart = pwv.astype(f32) * bin_w + sw + offx * _TRANS_STD * roi_w
    hstart = phv.astype(f32) * bin_h + sh + offy * _TRANS_STD * roi_h

    def _inrange(v, lim):
        return (v >= -0.5) & (v <= lim - 0.5)

    cw = sum(_inrange(wstart + float(j) * sub_w, W).astype(f32)
             for j in range(_S))
    ch = sum(_inrange(hstart + float(j) * sub_h, H).astype(f32)
             for j in range(_S))
    scale = 1.0 / jnp.maximum(cw * ch, 1.0)

    full = (NP, _BINS)
    cols = [wstart, hstart,
            jnp.broadcast_to(sub_w, full), jnp.broadcast_to(sub_h, full),
            scale, jnp.broadcast_to(bi * (H * W), full)]
    cols += [jnp.zeros(full, f32)] * (_L - len(cols))
    return jnp.stack(cols, axis=-1).reshape(NP * _BINS * _L)


def _sc_pool(tab, meta_flat, n_bins_padded, C, H, W):
    bpw = n_bins_padded // _NW
    chunk = 32
    nchunk = bpw // chunk
    ncc = C // _L
    _D = 4
    mesh = plsc.VectorSubcoreMesh(core_axis_name="c", subcore_axis_name="s",
                                  num_cores=_NC, num_subcores=_NS)

    @functools.partial(
        pl.kernel,
        mesh=mesh,
        out_type=jax.ShapeDtypeStruct((n_bins_padded * C,), jnp.float32),
        scratch_types=[
            pltpu.VMEM((chunk * _L,), jnp.float32),
            pltpu.VMEM((chunk * _SAMPLES,), jnp.int32),
            pltpu.VMEM((chunk * _SAMPLES * 4,), jnp.float32),
            pltpu.VMEM((_D, _SAMPLES, 4 * C), jnp.float32),
            pltpu.VMEM((_D, C), jnp.float32),
        ] + [pltpu.SemaphoreType.DMA] * (2 * _D),
    )
    def body(tab_hbm, meta_hbm, out_hbm, meta_v, idx_v, wts_v, rows_v,
             out_v, *sems):
        wid = lax.axis_index("s") * _NC + lax.axis_index("c")
        bin0 = wid * bpw
        gsems = sems[:_D]
        osems = sems[_D:]

        lane = lax.iota(jnp.int32, _L)
        offw = jnp.bitwise_and(lane, _S - 1).astype(jnp.float32)
        offh = jnp.right_shift(lane, 2).astype(jnp.float32)

        def gather(b, buf):
            return pltpu.async_copy(
                tab_hbm.at[idx_v.at[pl.ds(b * _SAMPLES, _SAMPLES)]],
                rows_v.at[buf], gsems[buf])

        def wait_gather(buf):
            pltpu.make_async_copy(
                tab_hbm.at[idx_v.at[pl.ds(0, _SAMPLES)]],
                rows_v.at[buf], gsems[buf]).wait()

        def prep(bb):
            mv = meta_v[pl.ds(bb * _L, _L)]

            def sp(i):
                return lax.broadcast(mv[i], (_L,))

            w = sp(0) + offw * sp(2)
            h = sp(1) + offh * sp(3)
            ok = ((w >= -0.5) & (w <= W - 0.5)
                  & (h >= -0.5) & (h <= H - 0.5))
            scale = jnp.where(ok, sp(4), 0.0)
            wc = jnp.minimum(jnp.maximum(w, 0.0), W - 1.0)
            hc = jnp.minimum(jnp.maximum(h, 0.0), H - 1.0)
            x0 = wc.astype(jnp.int32).astype(jnp.float32)
            y0 = hc.astype(jnp.int32).astype(jnp.float32)
            dx = wc - x0
            dy = hc - y0
            idxf = sp(5) + y0 * float(W) + x0
            idx_v[pl.ds(bb * _SAMPLES, _SAMPLES)] = idxf.astype(jnp.int32)
            wx1 = dx * scale
            wx0 = scale - wx1
            wts_v[pl.ds(bb * 64, _L)] = wx0 * (1.0 - dy)
            wts_v[pl.ds(bb * 64 + 16, _L)] = wx1 * (1.0 - dy)
            wts_v[pl.ds(bb * 64 + 32, _L)] = wx0 * dy
            wts_v[pl.ds(bb * 64 + 48, _L)] = wx1 * dy

        def accumulate(cb, b, buf, seen_prior):
            @pl.when(seen_prior)
            def _():
                pltpu.make_async_copy(
                    out_v.at[buf], out_hbm.at[pl.ds(0, C)],
                    osems[buf]).wait()
            zero = jnp.zeros((_L,), jnp.float32)
            for c in range(ncc):
                out_v[buf, pl.ds(c * _L, _L)] = zero

            @pl.loop(0, 4)
            def _grp(q):
                wvec = wts_v[pl.ds(b * (_SAMPLES * 4) + q * _L, _L)]
                wsps = [lax.broadcast(wvec[k], (_L,)) for k in range(16)]
                qoff = q * C

                def term(k, c):
                    return wsps[k] * rows_v[
                        buf, k, pl.ds(qoff + c * _L, _L)]

                for c0 in range(0, ncc, 2):
                    subs = [[term(k, c0 + h) for k in range(4)]
                            for h in range(2)]
                    for k in range(4, 16):
                        for h in range(2):
                            subs[h][k % 4] = subs[h][k % 4] + term(
                                k, c0 + h)
                    for h in range(2):
                        acc = (subs[h][0] + subs[h][1]) + (
                            subs[h][2] + subs[h][3])
                        plsc.addupdate(
                            out_v.at[buf, pl.ds((c0 + h) * _L, _L)], acc)
            pltpu.async_copy(out_v.at[buf],
                             out_hbm.at[pl.ds((cb + b) * C, C)], osems[buf])

        nq = chunk // _D

        @pl.loop(0, nchunk)
        def _chunk_loop(chi):
            cb = bin0 + chi * chunk
            pltpu.sync_copy(meta_hbm.at[pl.ds(cb * _L, chunk * _L)], meta_v)

            @pl.loop(0, chunk)
            def _prep_loop(bb):
                prep(bb)

            for buf in range(_D - 1):
                gather(buf, buf)

            @pl.loop(0, nq)
            def _quad_loop(p):
                seen = jnp.logical_or(chi > 0, p > 0)
                b = p * _D
                gather(b + _D - 1, _D - 1)
                for u in range(_D):
                    wait_gather(u)
                    accumulate(cb, b + u, u, seen)
                    if u < _D - 1:
                        @pl.when(p < nq - 1)
                        def _():
                            gather(b + _D + u, u)

        for buf in range(_D):
            pltpu.make_async_copy(out_v.at[buf], out_hbm.at[pl.ds(0, C)],
                                  osems[buf]).wait()

    return body(tab, meta_flat)


def kernel(input, rois, offset):
    B, C, H, W = input.shape
    N = rois.shape[0]
    NP = ((N + _ROW_BLK - 1) // _ROW_BLK) * _ROW_BLK
    nbins = NP * _BINS
    V = B * H * W

    xt = jnp.transpose(input, (0, 2, 3, 1)).reshape(V, C)
    xtp = jnp.pad(xt, ((0, W + 2), (0, 0)))
    tab = jnp.concatenate(
        [xtp[0:V], xtp[1:V + 1], xtp[W:V + W], xtp[W + 1:V + W + 1]], axis=1)

    pr = np.arange(_P, dtype=np.float32)
    part = np.floor(pr / np.float32(_P) * np.float32(_PART_SIZE)).astype(np.int64)
    off_x = offset[:, 0][:, part, :][:, :, part].reshape(N, _BINS)
    off_y = offset[:, 1][:, part, :][:, :, part].reshape(N, _BINS)

    rois_p = jnp.pad(rois, ((0, NP - N), (0, 0)))
    offx = jnp.pad(off_x, ((0, NP - N), (0, 0)))
    offy = jnp.pad(off_y, ((0, NP - N), (0, 0)))

    meta_flat = _bin_meta(rois_p, offx, offy, H, W)
    out_flat = _sc_pool(tab, meta_flat, nbins, C, H, W)

    out = out_flat.reshape(NP, _BINS, C)[:N]
    out = out.reshape(N, _P, _P, C)
    return jnp.transpose(out, (0, 3, 1, 2))
